# baseline (speedup 1.0000x reference)
import functools
import math

import jax
import jax.numpy as jnp
from jax import lax
from jax.experimental import pallas as pl
from jax.experimental.pallas import tpu as pltpu
from jax.experimental.pallas import tpu_sc as plsc

N = 2048
D = 256
H = 8
HD = D // H
E = 32768
B = 8

NC = 2
NS = 16
NW = NC * NS
G = 8
DG = D // G
TPG = NW // G
EPT = E // TPG
EDGE_CHUNK = 128
NCH = EPT // EDGE_CHUNK

TQ = 256


def _sc_scatter_add_body(xt_hbm, srcs_hbm, dsts_hbm, zeros_hbm, out_hbm,
                         idx_s, idx_d, rows, acc, sem):
    c = lax.axis_index("c")
    s = lax.axis_index("s")
    w = s * NC + c
    g = w // TPG
    t = w % TPG
    pltpu.sync_copy(zeros_hbm, acc)
    pltpu.sync_copy(srcs_hbm.at[g, t], idx_s)
    pltpu.sync_copy(dsts_hbm.at[t], idx_d)

    lane = lax.iota(jnp.int32, 16)

    def chunk(j, _):
        pltpu.async_copy(xt_hbm.at[idx_s.at[j]], rows, sem).wait()
        for k in range(EDGE_CHUNK // 16):
            dst_vec = idx_d[j, pl.ds(k * 16, 16)]
            erow = lane + (k * 16)
            for col in range(DG):
                cvec = jnp.full((16,), col, jnp.int32)
                vals = plsc.load_gather(rows, [erow, cvec])
                plsc.addupdate_scatter(acc, [dst_vec, cvec], vals)
        return _

    lax.fori_loop(0, NCH, chunk, None)
    pltpu.sync_copy(acc, out_hbm.at[t, g])


def _sc_scatter_add(xt, srcs, dsts, zeros):
    mesh = plsc.VectorSubcoreMesh(core_axis_name="c", subcore_axis_name="s")
    fn = pl.kernel(
        _sc_scatter_add_body,
        out_type=jax.ShapeDtypeStruct((TPG, G, N, DG), jnp.float32),
        mesh=mesh,
        compiler_params=pltpu.CompilerParams(
            needs_layout_passes=False, use_tc_tiling_on_sc=False),
        scratch_types=[
            pltpu.VMEM((NCH, EDGE_CHUNK), jnp.int32),
            pltpu.VMEM((NCH, EDGE_CHUNK), jnp.int32),
            pltpu.VMEM((EDGE_CHUNK, DG), jnp.float32),
            pltpu.VMEM((N, DG), jnp.float32),
            pltpu.SemaphoreType.DMA,
        ],
    )
    return fn(xt, srcs, dsts, zeros)


def _qkv_body(x_ref, w_ref, b_ref, out_ref):
    out_ref[...] = (
        jnp.dot(x_ref[...], w_ref[...], preferred_element_type=jnp.float32)
        + b_ref[...]
    )


def _qkv(x, Wqkv, bqkv_row):
    return pl.pallas_call(
        _qkv_body,
        out_shape=jax.ShapeDtypeStruct((N, 3 * D), jnp.float32),
    )(x, Wqkv, bqkv_row)


def _attn_body(q_ref, k_ref, v_ref, st_ref, en_ref, out_ref):
    col = lax.broadcasted_iota(jnp.int32, (TQ, N), 1)
    mask = (col >= st_ref[...]) & (col < en_ref[...])
    scale = jnp.float32(1.0 / math.sqrt(HD))
    q = q_ref[...]
    for h in range(H):
        sl = slice(h * HD, (h + 1) * HD)
        s = lax.dot_general(q[:, sl], k_ref[:, sl],
                            (((1,), (1,)), ((), ())),
                            preferred_element_type=jnp.float32) * scale
        s = jnp.where(mask, s, jnp.float32(-1e30))
        m = jnp.max(s, axis=1, keepdims=True)
        p = jnp.exp(s - m)
        l = jnp.sum(p, axis=1, keepdims=True)
        out_ref[:, sl] = lax.dot_general(p / l, v_ref[:, sl],
                                         (((1,), (0,)), ((), ())),
                                         preferred_element_type=jnp.float32)


def _attn(qkv, starts, ends):
    grid = (N // TQ,)
    return pl.pallas_call(
        _attn_body,
        grid=grid,
        in_specs=[
            pl.BlockSpec((TQ, D), lambda i: (i, 0)),
            pl.BlockSpec((N, D), lambda i: (0, 1)),
            pl.BlockSpec((N, D), lambda i: (0, 2)),
            pl.BlockSpec((TQ, 1), lambda i: (i, 0)),
            pl.BlockSpec((TQ, 1), lambda i: (i, 0)),
        ],
        out_specs=pl.BlockSpec((TQ, D), lambda i: (i, 0)),
        out_shape=jax.ShapeDtypeStruct((N, D), jnp.float32),
        compiler_params=pltpu.CompilerParams(
            dimension_semantics=("arbitrary",)),
    )(qkv, qkv, qkv, starts, ends)


def _bn(y, gamma, beta):
    mu = jnp.mean(y, axis=0, keepdims=True)
    var = jnp.mean((y - mu) * (y - mu), axis=0, keepdims=True)
    return (y - mu) / jnp.sqrt(var + 1e-5) * gamma + beta


def _tail_body(x_ref, agg_ref, ctx_ref, eps_ref,
               wg1_ref, bg1_ref, wg2_ref, bg2_ref,
               wo_ref, bo_ref, wf1_ref, bf1_ref, wf2_ref, bf2_ref,
               g1_ref, be1_ref, g2_ref, be2_ref, g3_ref, be3_ref, out_ref):
    x = x_ref[...]
    agg = jnp.concatenate(
        [sum(agg_ref[t, g] for t in range(TPG)) for g in range(G)], axis=1)
    z = (1.0 + eps_ref[0, 0]) * x + agg
    z = jnp.maximum(
        jnp.dot(z, wg1_ref[...], preferred_element_type=jnp.float32)
        + bg1_ref[...], 0.0)
    z = jnp.dot(z, wg2_ref[...], preferred_element_type=jnp.float32) + bg2_ref[...]
    h_local = _bn(x + z, g1_ref[...], be1_ref[...])
    proj = jnp.dot(ctx_ref[...], wo_ref[...],
                   preferred_element_type=jnp.float32) + bo_ref[...]
    h_attn = _bn(x + proj, g2_ref[...], be2_ref[...])
    h = h_local + h_attn
    ff = jnp.maximum(
        jnp.dot(h, wf1_ref[...], preferred_element_type=jnp.float32)
        + bf1_ref[...], 0.0)
    ff = jnp.dot(ff, wf2_ref[...], preferred_element_type=jnp.float32) + bf2_ref[...]
    out_ref[...] = _bn(h + ff, g3_ref[...], be3_ref[...])


def _tail(x, agg2, ctx, eps, Wg1, bg1, Wg2, bg2, Wo, bo,
          Wf1, bf1, Wf2, bf2, g1, be1, g2, be2, g3, be3):
    vmem = pl.BlockSpec(memory_space=pltpu.MemorySpace.VMEM)
    smem = pl.BlockSpec(memory_space=pltpu.MemorySpace.SMEM)
    return pl.pallas_call(
        _tail_body,
        in_specs=[vmem, vmem, vmem, smem] + [vmem] * 16,
        out_specs=vmem,
        out_shape=jax.ShapeDtypeStruct((N, D), jnp.float32),
    )(x, agg2, ctx, eps, Wg1, bg1, Wg2, bg2, Wo, bo,
      Wf1, bf1, Wf2, bf2, g1, be1, g2, be2, g3, be3)


def kernel(x, edge_index, batch_ids, Wg1, bg1, Wg2, bg2, eps_gin,
           Wqkv, bqkv, Wo, bo, Wf1, bf1, Wf2, bf2,
           gamma1, beta1, gamma2, beta2, gamma3, beta3):
    xt = x.reshape(N, G, DG).transpose(1, 0, 2).reshape(G * N, DG)
    src_q = edge_index[0].reshape(1, TPG, NCH, EDGE_CHUNK)
    srcs = src_q + (jnp.arange(G, dtype=jnp.int32) * N).reshape(G, 1, 1, 1)
    dsts = edge_index[1].reshape(TPG, NCH, EDGE_CHUNK)
    zeros = jnp.zeros((N, DG), jnp.float32)
    agg2 = _sc_scatter_add(xt, srcs, dsts, zeros)

    qkv = _qkv(x, Wqkv, bqkv.reshape(1, 3 * D))
    starts = jnp.searchsorted(batch_ids, batch_ids, side="left").astype(
        jnp.int32).reshape(N, 1)
    ends = jnp.searchsorted(batch_ids, batch_ids, side="right").astype(
        jnp.int32).reshape(N, 1)
    ctx = _attn(qkv, starts, ends)

    row = lambda v: v.reshape(1, -1)
    return _tail(x, agg2, ctx, eps_gin.reshape(1, 1),
                 Wg1, row(bg1), Wg2, row(bg2), Wo, row(bo),
                 Wf1, row(bf1), Wf2, row(bf2),
                 row(gamma1), row(beta1), row(gamma2), row(beta2),
                 row(gamma3), row(beta3))


# speedup vs baseline: 1.6130x; 1.6130x over previous
import functools
import math

import jax
import jax.numpy as jnp
from jax import lax
from jax.experimental import pallas as pl
from jax.experimental.pallas import tpu as pltpu
from jax.experimental.pallas import tpu_sc as plsc

N = 2048
D = 256
H = 8
HD = D // H
E = 32768
B = 8

NC = 2
NS = 16
NW = NC * NS
G = 8
DG = D // G
TPG = NW // G
EPT = E // TPG
EDGE_CHUNK = 128
NCH = EPT // EDGE_CHUNK

TQ = 256


def _sc_scatter_add_body(xt_hbm, srcs_hbm, dsts_hbm, zeros_hbm, out_hbm,
                         idx_s, idx_d, rows, rows2, acc, sem, sem2):
    c = lax.axis_index("c")
    s = lax.axis_index("s")
    w = s * NC + c
    g = w // TPG
    t = w % TPG
    pltpu.sync_copy(zeros_hbm, acc)
    pltpu.sync_copy(srcs_hbm.at[g, t], idx_s)
    pltpu.sync_copy(dsts_hbm.at[t], idx_d)

    def accumulate(buf, j):
        for k in range(EDGE_CHUNK // 16):
            dvec = idx_d[j, pl.ds(k * 16, 16)]
            for i in range(16):
                d = dvec[i]
                e = k * 16 + i
                for h in range(DG // 16):
                    plsc.addupdate(acc.at[d, pl.ds(h * 16, 16)],
                                   buf[e, pl.ds(h * 16, 16)])

    def gather(j, buf, s):
        return pltpu.async_copy(xt_hbm.at[idx_s.at[j]], buf, s)

    gather(0, rows, sem).wait()

    def two_chunks(jj, _):
        j0 = jj * 2
        cp1 = gather(j0 + 1, rows2, sem2)
        accumulate(rows, j0)
        cp1.wait()
        cp2 = gather(lax.rem(j0 + 2, NCH), rows, sem)
        accumulate(rows2, j0 + 1)
        cp2.wait()
        return _

    lax.fori_loop(0, NCH // 2, two_chunks, None)
    pltpu.sync_copy(acc, out_hbm.at[t, g])


def _sc_scatter_add(xt, srcs, dsts, zeros):
    mesh = plsc.VectorSubcoreMesh(core_axis_name="c", subcore_axis_name="s")
    fn = pl.kernel(
        _sc_scatter_add_body,
        out_type=jax.ShapeDtypeStruct((TPG, G, N, DG), jnp.float32),
        mesh=mesh,
        compiler_params=pltpu.CompilerParams(
            needs_layout_passes=False, use_tc_tiling_on_sc=False),
        scratch_types=[
            pltpu.VMEM((NCH, EDGE_CHUNK), jnp.int32),
            pltpu.VMEM((NCH, EDGE_CHUNK), jnp.int32),
            pltpu.VMEM((EDGE_CHUNK, DG), jnp.float32),
            pltpu.VMEM((EDGE_CHUNK, DG), jnp.float32),
            pltpu.VMEM((N, DG), jnp.float32),
            pltpu.SemaphoreType.DMA,
            pltpu.SemaphoreType.DMA,
        ],
    )
    return fn(xt, srcs, dsts, zeros)


def _qkv_body(x_ref, w_ref, b_ref, out_ref):
    out_ref[...] = (
        jnp.dot(x_ref[...], w_ref[...], preferred_element_type=jnp.float32)
        + b_ref[...]
    )


def _qkv(x, Wqkv, bqkv_row):
    return pl.pallas_call(
        _qkv_body,
        out_shape=jax.ShapeDtypeStruct((N, 3 * D), jnp.float32),
    )(x, Wqkv, bqkv_row)


def _attn_body(q_ref, k_ref, v_ref, st_ref, en_ref, out_ref):
    col = lax.broadcasted_iota(jnp.int32, (TQ, N), 1)
    mask = (col >= st_ref[...]) & (col < en_ref[...])
    scale = jnp.float32(1.0 / math.sqrt(HD))
    q = q_ref[...]
    for h in range(H):
        sl = slice(h * HD, (h + 1) * HD)
        s = lax.dot_general(q[:, sl], k_ref[:, sl],
                            (((1,), (1,)), ((), ())),
                            preferred_element_type=jnp.float32) * scale
        s = jnp.where(mask, s, jnp.float32(-1e30))
        m = jnp.max(s, axis=1, keepdims=True)
        p = jnp.exp(s - m)
        l = jnp.sum(p, axis=1, keepdims=True)
        out_ref[:, sl] = lax.dot_general(p / l, v_ref[:, sl],
                                         (((1,), (0,)), ((), ())),
                                         preferred_element_type=jnp.float32)


def _attn(qkv, starts, ends):
    grid = (N // TQ,)
    return pl.pallas_call(
        _attn_body,
        grid=grid,
        in_specs=[
            pl.BlockSpec((TQ, D), lambda i: (i, 0)),
            pl.BlockSpec((N, D), lambda i: (0, 1)),
            pl.BlockSpec((N, D), lambda i: (0, 2)),
            pl.BlockSpec((TQ, 1), lambda i: (i, 0)),
            pl.BlockSpec((TQ, 1), lambda i: (i, 0)),
        ],
        out_specs=pl.BlockSpec((TQ, D), lambda i: (i, 0)),
        out_shape=jax.ShapeDtypeStruct((N, D), jnp.float32),
        compiler_params=pltpu.CompilerParams(
            dimension_semantics=("arbitrary",)),
    )(qkv, qkv, qkv, starts, ends)


def _bn(y, gamma, beta):
    mu = jnp.mean(y, axis=0, keepdims=True)
    var = jnp.mean((y - mu) * (y - mu), axis=0, keepdims=True)
    return (y - mu) / jnp.sqrt(var + 1e-5) * gamma + beta


def _tail_body(x_ref, agg_ref, ctx_ref, eps_ref,
               wg1_ref, bg1_ref, wg2_ref, bg2_ref,
               wo_ref, bo_ref, wf1_ref, bf1_ref, wf2_ref, bf2_ref,
               g1_ref, be1_ref, g2_ref, be2_ref, g3_ref, be3_ref, out_ref):
    x = x_ref[...]
    agg = jnp.concatenate(
        [sum(agg_ref[t, g] for t in range(TPG)) for g in range(G)], axis=1)
    z = (1.0 + eps_ref[0, 0]) * x + agg
    z = jnp.maximum(
        jnp.dot(z, wg1_ref[...], preferred_element_type=jnp.float32)
        + bg1_ref[...], 0.0)
    z = jnp.dot(z, wg2_ref[...], preferred_element_type=jnp.float32) + bg2_ref[...]
    h_local = _bn(x + z, g1_ref[...], be1_ref[...])
    proj = jnp.dot(ctx_ref[...], wo_ref[...],
                   preferred_element_type=jnp.float32) + bo_ref[...]
    h_attn = _bn(x + proj, g2_ref[...], be2_ref[...])
    h = h_local + h_attn
    ff = jnp.maximum(
        jnp.dot(h, wf1_ref[...], preferred_element_type=jnp.float32)
        + bf1_ref[...], 0.0)
    ff = jnp.dot(ff, wf2_ref[...], preferred_element_type=jnp.float32) + bf2_ref[...]
    out_ref[...] = _bn(h + ff, g3_ref[...], be3_ref[...])


def _tail(x, agg2, ctx, eps, Wg1, bg1, Wg2, bg2, Wo, bo,
          Wf1, bf1, Wf2, bf2, g1, be1, g2, be2, g3, be3):
    vmem = pl.BlockSpec(memory_space=pltpu.MemorySpace.VMEM)
    smem = pl.BlockSpec(memory_space=pltpu.MemorySpace.SMEM)
    return pl.pallas_call(
        _tail_body,
        in_specs=[vmem, vmem, vmem, smem] + [vmem] * 16,
        out_specs=vmem,
        out_shape=jax.ShapeDtypeStruct((N, D), jnp.float32),
    )(x, agg2, ctx, eps, Wg1, bg1, Wg2, bg2, Wo, bo,
      Wf1, bf1, Wf2, bf2, g1, be1, g2, be2, g3, be3)


def kernel(x, edge_index, batch_ids, Wg1, bg1, Wg2, bg2, eps_gin,
           Wqkv, bqkv, Wo, bo, Wf1, bf1, Wf2, bf2,
           gamma1, beta1, gamma2, beta2, gamma3, beta3):
    xt = x.reshape(N, G, DG).transpose(1, 0, 2).reshape(G * N, DG)
    src_q = edge_index[0].reshape(1, TPG, NCH, EDGE_CHUNK)
    srcs = src_q + (jnp.arange(G, dtype=jnp.int32) * N).reshape(G, 1, 1, 1)
    dsts = edge_index[1].reshape(TPG, NCH, EDGE_CHUNK)
    zeros = jnp.zeros((N, DG), jnp.float32)
    agg2 = _sc_scatter_add(xt, srcs, dsts, zeros)

    qkv = _qkv(x, Wqkv, bqkv.reshape(1, 3 * D))
    starts = jnp.searchsorted(batch_ids, batch_ids, side="left").astype(
        jnp.int32).reshape(N, 1)
    ends = jnp.searchsorted(batch_ids, batch_ids, side="right").astype(
        jnp.int32).reshape(N, 1)
    ctx = _attn(qkv, starts, ends)

    row = lambda v: v.reshape(1, -1)
    return _tail(x, agg2, ctx, eps_gin.reshape(1, 1),
                 Wg1, row(bg1), Wg2, row(bg2), Wo, row(bo),
                 Wf1, row(bf1), Wf2, row(bf2),
                 row(gamma1), row(beta1), row(gamma2), row(beta2),
                 row(gamma3), row(beta3))


# speedup vs baseline: 4.2170x; 2.6144x over previous
import functools
import math

import jax
import jax.numpy as jnp
from jax import lax
from jax.experimental import pallas as pl
from jax.experimental.pallas import tpu as pltpu
from jax.experimental.pallas import tpu_sc as plsc

N = 2048
D = 256
H = 8
HD = D // H
E = 32768
B = 8

NC = 2
NS = 16
NW = NC * NS
G = 8
DG = D // G
TPG = NW // G
EPT = E // TPG
EDGE_CHUNK = 128
NCH = EPT // EDGE_CHUNK

TQ = 256


def _sc_scatter_add_body(xt_hbm, srcs_hbm, dsts_hbm, zeros_hbm, out_hbm,
                         idx_s, idx_d, rows, rows2, acc, sem, sem2):
    c = lax.axis_index("c")
    s = lax.axis_index("s")
    w = s * NC + c
    g = w // TPG
    t = w % TPG
    pltpu.sync_copy(zeros_hbm, acc)
    pltpu.sync_copy(srcs_hbm.at[g, t], idx_s)
    pltpu.sync_copy(dsts_hbm.at[t], idx_d)

    def accumulate(buf, j):
        for k in range(EDGE_CHUNK // 16):
            dvec = idx_d[j, pl.ds(k * 16, 16)]
            for i in range(16):
                d = dvec[i]
                e = k * 16 + i
                for h in range(DG // 16):
                    plsc.addupdate(acc.at[d, pl.ds(h * 16, 16)],
                                   buf[e, pl.ds(h * 16, 16)])

    def gather(j, buf, s):
        return pltpu.async_copy(xt_hbm.at[idx_s.at[j]], buf, s)

    gather(0, rows, sem).wait()

    def two_chunks(jj, _):
        j0 = jj * 2
        cp1 = gather(j0 + 1, rows2, sem2)
        accumulate(rows, j0)
        cp1.wait()
        cp2 = gather(lax.rem(j0 + 2, NCH), rows, sem)
        accumulate(rows2, j0 + 1)
        cp2.wait()
        return _

    lax.fori_loop(0, NCH // 2, two_chunks, None)
    pltpu.sync_copy(acc, out_hbm.at[t, g])


def _sc_scatter_add(xt, srcs, dsts, zeros):
    mesh = plsc.VectorSubcoreMesh(core_axis_name="c", subcore_axis_name="s")
    fn = pl.kernel(
        _sc_scatter_add_body,
        out_type=jax.ShapeDtypeStruct((TPG, G, N, DG), jnp.float32),
        mesh=mesh,
        compiler_params=pltpu.CompilerParams(
            needs_layout_passes=False, use_tc_tiling_on_sc=False),
        scratch_types=[
            pltpu.VMEM((NCH, EDGE_CHUNK), jnp.int32),
            pltpu.VMEM((NCH, EDGE_CHUNK), jnp.int32),
            pltpu.VMEM((EDGE_CHUNK, DG), jnp.float32),
            pltpu.VMEM((EDGE_CHUNK, DG), jnp.float32),
            pltpu.VMEM((N, DG), jnp.float32),
            pltpu.SemaphoreType.DMA,
            pltpu.SemaphoreType.DMA,
        ],
    )
    return fn(xt, srcs, dsts, zeros)


def _qkv_body(x_ref, w_ref, b_ref, out_ref):
    out_ref[...] = (
        jnp.dot(x_ref[...], w_ref[...], preferred_element_type=jnp.float32)
        + b_ref[...]
    )


def _qkv(x, Wqkv, bqkv_row):
    return pl.pallas_call(
        _qkv_body,
        out_shape=jax.ShapeDtypeStruct((N, 3 * D), jnp.float32),
    )(x, Wqkv, bqkv_row)


def _attn_body(q_ref, k_ref, v_ref, bq_ref, br_ref, out_ref):
    mask = bq_ref[...] == br_ref[0:1, :]
    scale = jnp.float32(1.0 / math.sqrt(HD))
    q = q_ref[...]
    for h in range(H):
        sl = slice(h * HD, (h + 1) * HD)
        s = lax.dot_general(q[:, sl], k_ref[:, sl],
                            (((1,), (1,)), ((), ())),
                            preferred_element_type=jnp.float32) * scale
        s = jnp.where(mask, s, jnp.float32(-1e30))
        m = jnp.max(s, axis=1, keepdims=True)
        p = jnp.exp(s - m)
        l = jnp.sum(p, axis=1, keepdims=True)
        out_ref[:, sl] = lax.dot_general(p / l, v_ref[:, sl],
                                         (((1,), (0,)), ((), ())),
                                         preferred_element_type=jnp.float32)


def _attn(qkv, bq, br):
    grid = (N // TQ,)
    return pl.pallas_call(
        _attn_body,
        grid=grid,
        in_specs=[
            pl.BlockSpec((TQ, D), lambda i: (i, 0)),
            pl.BlockSpec((N, D), lambda i: (0, 1)),
            pl.BlockSpec((N, D), lambda i: (0, 2)),
            pl.BlockSpec((TQ, 1), lambda i: (i, 0)),
            pl.BlockSpec((8, N), lambda i: (0, 0)),
        ],
        out_specs=pl.BlockSpec((TQ, D), lambda i: (i, 0)),
        out_shape=jax.ShapeDtypeStruct((N, D), jnp.float32),
        compiler_params=pltpu.CompilerParams(
            dimension_semantics=("arbitrary",)),
    )(qkv, qkv, qkv, bq, br)


def _bn(y, gamma, beta):
    mu = jnp.mean(y, axis=0, keepdims=True)
    var = jnp.mean((y - mu) * (y - mu), axis=0, keepdims=True)
    return (y - mu) / jnp.sqrt(var + 1e-5) * gamma + beta


def _tail_body(x_ref, agg_ref, ctx_ref, eps_ref,
               wg1_ref, bg1_ref, wg2_ref, bg2_ref,
               wo_ref, bo_ref, wf1_ref, bf1_ref, wf2_ref, bf2_ref,
               g1_ref, be1_ref, g2_ref, be2_ref, g3_ref, be3_ref, out_ref):
    x = x_ref[...]
    agg = jnp.concatenate(
        [sum(agg_ref[t, g] for t in range(TPG)) for g in range(G)], axis=1)
    z = (1.0 + eps_ref[0, 0]) * x + agg
    z = jnp.maximum(
        jnp.dot(z, wg1_ref[...], preferred_element_type=jnp.float32)
        + bg1_ref[...], 0.0)
    z = jnp.dot(z, wg2_ref[...], preferred_element_type=jnp.float32) + bg2_ref[...]
    h_local = _bn(x + z, g1_ref[...], be1_ref[...])
    proj = jnp.dot(ctx_ref[...], wo_ref[...],
                   preferred_element_type=jnp.float32) + bo_ref[...]
    h_attn = _bn(x + proj, g2_ref[...], be2_ref[...])
    h = h_local + h_attn
    ff = jnp.maximum(
        jnp.dot(h, wf1_ref[...], preferred_element_type=jnp.float32)
        + bf1_ref[...], 0.0)
    ff = jnp.dot(ff, wf2_ref[...], preferred_element_type=jnp.float32) + bf2_ref[...]
    out_ref[...] = _bn(h + ff, g3_ref[...], be3_ref[...])


def _tail(x, agg2, ctx, eps, Wg1, bg1, Wg2, bg2, Wo, bo,
          Wf1, bf1, Wf2, bf2, g1, be1, g2, be2, g3, be3):
    vmem = pl.BlockSpec(memory_space=pltpu.MemorySpace.VMEM)
    smem = pl.BlockSpec(memory_space=pltpu.MemorySpace.SMEM)
    return pl.pallas_call(
        _tail_body,
        in_specs=[vmem, vmem, vmem, smem] + [vmem] * 16,
        out_specs=vmem,
        out_shape=jax.ShapeDtypeStruct((N, D), jnp.float32),
    )(x, agg2, ctx, eps, Wg1, bg1, Wg2, bg2, Wo, bo,
      Wf1, bf1, Wf2, bf2, g1, be1, g2, be2, g3, be3)


def kernel(x, edge_index, batch_ids, Wg1, bg1, Wg2, bg2, eps_gin,
           Wqkv, bqkv, Wo, bo, Wf1, bf1, Wf2, bf2,
           gamma1, beta1, gamma2, beta2, gamma3, beta3):
    xt = x.reshape(N, G, DG).transpose(1, 0, 2).reshape(G * N, DG)
    src_q = edge_index[0].reshape(1, TPG, NCH, EDGE_CHUNK)
    srcs = src_q + (jnp.arange(G, dtype=jnp.int32) * N).reshape(G, 1, 1, 1)
    dsts = edge_index[1].reshape(TPG, NCH, EDGE_CHUNK)
    zeros = jnp.zeros((N, DG), jnp.float32)
    agg2 = _sc_scatter_add(xt, srcs, dsts, zeros)

    qkv = _qkv(x, Wqkv, bqkv.reshape(1, 3 * D))
    bq = batch_ids.reshape(N, 1)
    br = jnp.broadcast_to(batch_ids.reshape(1, N), (8, N))
    ctx = _attn(qkv, bq, br)

    row = lambda v: v.reshape(1, -1)
    return _tail(x, agg2, ctx, eps_gin.reshape(1, 1),
                 Wg1, row(bg1), Wg2, row(bg2), Wo, row(bo),
                 Wf1, row(bf1), Wf2, row(bf2),
                 row(gamma1), row(beta1), row(gamma2), row(beta2),
                 row(gamma3), row(beta3))


# speedup vs baseline: 4.5191x; 1.0716x over previous
import functools
import math

import jax
import jax.numpy as jnp
from jax import lax
from jax.experimental import pallas as pl
from jax.experimental.pallas import tpu as pltpu
from jax.experimental.pallas import tpu_sc as plsc

N = 2048
D = 256
H = 8
HD = D // H
E = 32768
B = 8

NC = 2
NS = 16
NW = NC * NS
G = 8
DG = D // G
TPG = NW // G
EPT = E // TPG
EDGE_CHUNK = 128
NCH = EPT // EDGE_CHUNK

TQ = 256


def _sc_scatter_add_body(xt_hbm, srcs_hbm, dsts_hbm, zeros_hbm, out_hbm,
                         idx_s, idx_d, rows, rows2, acc, sem, sem2):
    c = lax.axis_index("c")
    s = lax.axis_index("s")
    w = s * NC + c
    g = w // TPG
    t = w % TPG
    pltpu.sync_copy(zeros_hbm, acc)
    pltpu.sync_copy(srcs_hbm.at[g, t], idx_s)
    pltpu.sync_copy(dsts_hbm.at[t], idx_d)

    def accumulate(buf, j):
        for k in range(EDGE_CHUNK // 16):
            dvec = idx_d[j, pl.ds(k * 16, 16)]
            ds = [dvec[i] for i in range(16)]
            for i in range(16):
                e = k * 16 + i
                vals = [buf[e, pl.ds(h * 16, 16)] for h in range(DG // 16)]
                for h in range(DG // 16):
                    plsc.addupdate(acc.at[ds[i], pl.ds(h * 16, 16)], vals[h])

    def gather(j, buf, s):
        return pltpu.async_copy(xt_hbm.at[idx_s.at[j]], buf, s)

    gather(0, rows, sem).wait()

    def two_chunks(jj, _):
        j0 = jj * 2
        cp1 = gather(j0 + 1, rows2, sem2)
        accumulate(rows, j0)
        cp1.wait()
        cp2 = gather(lax.rem(j0 + 2, NCH), rows, sem)
        accumulate(rows2, j0 + 1)
        cp2.wait()
        return _

    lax.fori_loop(0, NCH // 2, two_chunks, None)
    pltpu.sync_copy(acc, out_hbm.at[t, g])


def _sc_scatter_add(xt, srcs, dsts, zeros):
    mesh = plsc.VectorSubcoreMesh(core_axis_name="c", subcore_axis_name="s")
    fn = pl.kernel(
        _sc_scatter_add_body,
        out_type=jax.ShapeDtypeStruct((TPG, G, N, DG), jnp.float32),
        mesh=mesh,
        compiler_params=pltpu.CompilerParams(
            needs_layout_passes=False, use_tc_tiling_on_sc=False),
        scratch_types=[
            pltpu.VMEM((NCH, EDGE_CHUNK), jnp.int32),
            pltpu.VMEM((NCH, EDGE_CHUNK), jnp.int32),
            pltpu.VMEM((EDGE_CHUNK, DG), jnp.float32),
            pltpu.VMEM((EDGE_CHUNK, DG), jnp.float32),
            pltpu.VMEM((N, DG), jnp.float32),
            pltpu.SemaphoreType.DMA,
            pltpu.SemaphoreType.DMA,
        ],
    )
    return fn(xt, srcs, dsts, zeros)


def _qkv_body(x_ref, w_ref, b_ref, out_ref):
    out_ref[...] = (
        jnp.dot(x_ref[...], w_ref[...], preferred_element_type=jnp.float32)
        + b_ref[...]
    )


def _qkv(x, Wqkv, bqkv_row):
    return pl.pallas_call(
        _qkv_body,
        out_shape=jax.ShapeDtypeStruct((N, 3 * D), jnp.float32),
    )(x, Wqkv, bqkv_row)


def _attn_body(q_ref, k_ref, v_ref, bq_ref, br_ref, out_ref):
    mask = bq_ref[...] == br_ref[0:1, :]
    scale = jnp.float32(1.0 / math.sqrt(HD))
    q = q_ref[...]
    for h in range(H):
        sl = slice(h * HD, (h + 1) * HD)
        s = lax.dot_general(q[:, sl], k_ref[:, sl],
                            (((1,), (1,)), ((), ())),
                            preferred_element_type=jnp.float32) * scale
        s = jnp.where(mask, s, jnp.float32(-1e30))
        m = jnp.max(s, axis=1, keepdims=True)
        p = jnp.exp(s - m)
        l = jnp.sum(p, axis=1, keepdims=True)
        out_ref[:, sl] = lax.dot_general(p / l, v_ref[:, sl],
                                         (((1,), (0,)), ((), ())),
                                         preferred_element_type=jnp.float32)


def _attn(qkv, bq, br):
    grid = (N // TQ,)
    return pl.pallas_call(
        _attn_body,
        grid=grid,
        in_specs=[
            pl.BlockSpec((TQ, D), lambda i: (i, 0)),
            pl.BlockSpec((N, D), lambda i: (0, 1)),
            pl.BlockSpec((N, D), lambda i: (0, 2)),
            pl.BlockSpec((TQ, 1), lambda i: (i, 0)),
            pl.BlockSpec((8, N), lambda i: (0, 0)),
        ],
        out_specs=pl.BlockSpec((TQ, D), lambda i: (i, 0)),
        out_shape=jax.ShapeDtypeStruct((N, D), jnp.float32),
        compiler_params=pltpu.CompilerParams(
            dimension_semantics=("arbitrary",)),
    )(qkv, qkv, qkv, bq, br)


def _bn(y, gamma, beta):
    mu = jnp.mean(y, axis=0, keepdims=True)
    var = jnp.mean((y - mu) * (y - mu), axis=0, keepdims=True)
    return (y - mu) / jnp.sqrt(var + 1e-5) * gamma + beta


def _tail_body(x_ref, agg_ref, ctx_ref, eps_ref,
               wg1_ref, bg1_ref, wg2_ref, bg2_ref,
               wo_ref, bo_ref, wf1_ref, bf1_ref, wf2_ref, bf2_ref,
               g1_ref, be1_ref, g2_ref, be2_ref, g3_ref, be3_ref, out_ref):
    x = x_ref[...]
    agg = jnp.concatenate(
        [sum(agg_ref[t, g] for t in range(TPG)) for g in range(G)], axis=1)
    z = (1.0 + eps_ref[0, 0]) * x + agg
    z = jnp.maximum(
        jnp.dot(z, wg1_ref[...], preferred_element_type=jnp.float32)
        + bg1_ref[...], 0.0)
    z = jnp.dot(z, wg2_ref[...], preferred_element_type=jnp.float32) + bg2_ref[...]
    h_local = _bn(x + z, g1_ref[...], be1_ref[...])
    proj = jnp.dot(ctx_ref[...], wo_ref[...],
                   preferred_element_type=jnp.float32) + bo_ref[...]
    h_attn = _bn(x + proj, g2_ref[...], be2_ref[...])
    h = h_local + h_attn
    ff = jnp.maximum(
        jnp.dot(h, wf1_ref[...], preferred_element_type=jnp.float32)
        + bf1_ref[...], 0.0)
    ff = jnp.dot(ff, wf2_ref[...], preferred_element_type=jnp.float32) + bf2_ref[...]
    out_ref[...] = _bn(h + ff, g3_ref[...], be3_ref[...])


def _tail(x, agg2, ctx, eps, Wg1, bg1, Wg2, bg2, Wo, bo,
          Wf1, bf1, Wf2, bf2, g1, be1, g2, be2, g3, be3):
    vmem = pl.BlockSpec(memory_space=pltpu.MemorySpace.VMEM)
    smem = pl.BlockSpec(memory_space=pltpu.MemorySpace.SMEM)
    return pl.pallas_call(
        _tail_body,
        in_specs=[vmem, vmem, vmem, smem] + [vmem] * 16,
        out_specs=vmem,
        out_shape=jax.ShapeDtypeStruct((N, D), jnp.float32),
    )(x, agg2, ctx, eps, Wg1, bg1, Wg2, bg2, Wo, bo,
      Wf1, bf1, Wf2, bf2, g1, be1, g2, be2, g3, be3)


def kernel(x, edge_index, batch_ids, Wg1, bg1, Wg2, bg2, eps_gin,
           Wqkv, bqkv, Wo, bo, Wf1, bf1, Wf2, bf2,
           gamma1, beta1, gamma2, beta2, gamma3, beta3):
    xt = x.reshape(N, G, DG).transpose(1, 0, 2).reshape(G * N, DG)
    src_q = edge_index[0].reshape(1, TPG, NCH, EDGE_CHUNK)
    srcs = src_q + (jnp.arange(G, dtype=jnp.int32) * N).reshape(G, 1, 1, 1)
    dsts = edge_index[1].reshape(TPG, NCH, EDGE_CHUNK)
    zeros = jnp.zeros((N, DG), jnp.float32)
    agg2 = _sc_scatter_add(xt, srcs, dsts, zeros)

    qkv = _qkv(x, Wqkv, bqkv.reshape(1, 3 * D))
    bq = batch_ids.reshape(N, 1)
    br = jnp.broadcast_to(batch_ids.reshape(1, N), (8, N))
    ctx = _attn(qkv, bq, br)

    row = lambda v: v.reshape(1, -1)
    return _tail(x, agg2, ctx, eps_gin.reshape(1, 1),
                 Wg1, row(bg1), Wg2, row(bg2), Wo, row(bo),
                 Wf1, row(bf1), Wf2, row(bf2),
                 row(gamma1), row(beta1), row(gamma2), row(beta2),
                 row(gamma3), row(beta3))
